# Initial kernel scaffold; baseline (speedup 1.0000x reference)
import jax
import jax.numpy as jnp
from jax import lax
from jax.experimental import pallas as pl
from jax.experimental.pallas import tpu as pltpu
from jax.experimental.pallas import tpu_sc as plsc

BATCH = 16384
SEQ_LEN = 50
DIM = 32

_INFO = plsc.get_sparse_core_info()
_NC = _INFO.num_cores
_NS = _INFO.num_subcores
_NW = _NC * _NS
_ROWS_PER_W = BATCH // _NW
_BLK_ROWS = 2
_BLK_IDX = _BLK_ROWS * SEQ_LEN
_NBLK = _ROWS_PER_W // _BLK_ROWS
_INV_L = float(1.0 / SEQ_LEN)


def _sc_kernel(x_hbm, table_hbm, out_hbm, idx_v, rows_v, out_v, sem):
    wid = lax.axis_index("s") * _NC + lax.axis_index("c")

    pltpu.sync_copy(x_hbm.at[wid], idx_v)

    def body(blk, _):
        pltpu.async_copy(table_hbm.at[idx_v.at[blk]], rows_v, sem).wait()
        for r in range(_BLK_ROWS):
            acc0 = jnp.zeros((16,), jnp.float32)
            acc1 = jnp.zeros((16,), jnp.float32)
            for j in range(SEQ_LEN):
                acc0 = acc0 + rows_v[r * SEQ_LEN + j, pl.ds(0, 16)]
                acc1 = acc1 + rows_v[r * SEQ_LEN + j, pl.ds(16, 16)]
            row = blk * _BLK_ROWS + r
            out_v[row, pl.ds(0, 16)] = acc0 * _INV_L
            out_v[row, pl.ds(16, 16)] = acc1 * _INV_L
        return ()

    lax.fori_loop(0, _NBLK, body, ())

    pltpu.sync_copy(out_v, out_hbm.at[pl.ds(wid * _ROWS_PER_W, _ROWS_PER_W)])


@jax.jit
def kernel(x, table):
    x_tiles = x.reshape(_NW, _NBLK, _BLK_IDX)
    mesh = plsc.VectorSubcoreMesh(core_axis_name="c", subcore_axis_name="s")
    run = pl.kernel(
        _sc_kernel,
        out_type=jax.ShapeDtypeStruct((BATCH, DIM), jnp.float32),
        mesh=mesh,
        scratch_types=[
            pltpu.VMEM((_NBLK, _BLK_IDX), jnp.int32),
            pltpu.VMEM((_BLK_IDX, DIM), jnp.float32),
            pltpu.VMEM((_ROWS_PER_W, DIM), jnp.float32),
            pltpu.SemaphoreType.DMA,
        ],
    )
    return run(x_tiles, table)


# speedup vs baseline: 2.2569x; 2.2569x over previous
import jax
import jax.numpy as jnp
from jax import lax
from jax.experimental import pallas as pl
from jax.experimental.pallas import tpu as pltpu
from jax.experimental.pallas import tpu_sc as plsc

BATCH = 16384
SEQ_LEN = 50
DIM = 32

_NC = 2
_NS = 16
_NW = _NC * _NS
_ROWS_PER_W = BATCH // _NW
_BLK_ROWS = 2
_BLK_IDX = _BLK_ROWS * SEQ_LEN
_NBLK = _ROWS_PER_W // _BLK_ROWS
_INV_L = float(1.0 / SEQ_LEN)


def _sc_kernel(x_hbm, table_hbm, out_hbm, idx_v, rows_v, out_v, sem):
    wid = lax.axis_index("s") * _NC + lax.axis_index("c")

    pltpu.sync_copy(x_hbm.at[wid], idx_v)

    def body(blk, _):
        pltpu.async_copy(table_hbm.at[idx_v.at[blk]], rows_v, sem).wait()
        for r in range(_BLK_ROWS):
            acc0 = jnp.zeros((16,), jnp.float32)
            acc1 = jnp.zeros((16,), jnp.float32)
            for j in range(SEQ_LEN):
                acc0 = acc0 + rows_v[r * SEQ_LEN + j, pl.ds(0, 16)]
                acc1 = acc1 + rows_v[r * SEQ_LEN + j, pl.ds(16, 16)]
            row = blk * _BLK_ROWS + r
            out_v[row, pl.ds(0, 16)] = acc0 * _INV_L
            out_v[row, pl.ds(16, 16)] = acc1 * _INV_L
        return ()

    lax.fori_loop(0, _NBLK, body, ())

    pltpu.sync_copy(out_v, out_hbm.at[pl.ds(wid * _ROWS_PER_W, _ROWS_PER_W)])


@jax.jit
def kernel(x, table):
    x_tiles = x.reshape(_NW, _NBLK, _BLK_IDX)
    mesh = plsc.VectorSubcoreMesh(
        core_axis_name="c", subcore_axis_name="s",
        num_cores=_NC, num_subcores=_NS,
    )
    run = pl.kernel(
        _sc_kernel,
        out_type=jax.ShapeDtypeStruct((BATCH, DIM), jnp.float32),
        mesh=mesh,
        scratch_types=[
            pltpu.VMEM((_NBLK, _BLK_IDX), jnp.int32),
            pltpu.VMEM((_BLK_IDX, DIM), jnp.float32),
            pltpu.VMEM((_ROWS_PER_W, DIM), jnp.float32),
            pltpu.SemaphoreType.DMA,
        ],
        compiler_params=pltpu.CompilerParams(use_tc_tiling_on_sc=False),
    )
    return run(x_tiles, table)


# speedup vs baseline: 2.9461x; 1.3054x over previous
import jax
import jax.numpy as jnp
from jax import lax
from jax.experimental import pallas as pl
from jax.experimental.pallas import tpu as pltpu
from jax.experimental.pallas import tpu_sc as plsc

BATCH = 16384
SEQ_LEN = 50
DIM = 32

_NC = 2
_NS = 16
_NW = _NC * _NS
_ROWS_PER_W = BATCH // _NW
_BLK_ROWS = 16
_BLK_IDX = _BLK_ROWS * SEQ_LEN
_NBLK = _ROWS_PER_W // _BLK_ROWS
_NBUF = 2
_NGRP = _NBLK // _NBUF
_INV_L = float(1.0 / SEQ_LEN)


def _sc_kernel(x_hbm, table_hbm, out_hbm, idx_v, rows_v, out_v, *sems):
    wid = lax.axis_index("s") * _NC + lax.axis_index("c")

    pltpu.sync_copy(x_hbm.at[wid], idx_v)

    for b in range(_NBUF):
        pltpu.async_copy(table_hbm.at[idx_v.at[b]], rows_v.at[b], sems[b])

    def reduce_row(b, blk):
        def body(r, _):
            acc0 = jnp.zeros((16,), jnp.float32)
            acc1 = jnp.zeros((16,), jnp.float32)
            for j in range(SEQ_LEN):
                acc0 = acc0 + rows_v[b, r * SEQ_LEN + j, pl.ds(0, 16)]
                acc1 = acc1 + rows_v[b, r * SEQ_LEN + j, pl.ds(16, 16)]
            row = blk * _BLK_ROWS + r
            out_v[row, pl.ds(0, 16)] = acc0 * _INV_L
            out_v[row, pl.ds(16, 16)] = acc1 * _INV_L
            return ()

        lax.fori_loop(0, _BLK_ROWS, body, ())

    def group(g, _):
        for b in range(_NBUF):
            blk = g * _NBUF + b
            pltpu.make_async_copy(
                table_hbm.at[idx_v.at[blk]], rows_v.at[b], sems[b]
            ).wait()
            reduce_row(b, blk)

            @pl.when(g < _NGRP - 1)
            def _():
                pltpu.async_copy(
                    table_hbm.at[idx_v.at[blk + _NBUF]], rows_v.at[b], sems[b]
                )

        return ()

    lax.fori_loop(0, _NGRP, group, ())

    pltpu.sync_copy(out_v, out_hbm.at[pl.ds(wid * _ROWS_PER_W, _ROWS_PER_W)])


@jax.jit
def kernel(x, table):
    x_tiles = x.reshape(_NW, _NBLK, _BLK_IDX)
    mesh = plsc.VectorSubcoreMesh(
        core_axis_name="c", subcore_axis_name="s",
        num_cores=_NC, num_subcores=_NS,
    )
    run = pl.kernel(
        _sc_kernel,
        out_type=jax.ShapeDtypeStruct((BATCH, DIM), jnp.float32),
        mesh=mesh,
        scratch_types=[
            pltpu.VMEM((_NBLK, _BLK_IDX), jnp.int32),
            pltpu.VMEM((_NBUF, _BLK_IDX, DIM), jnp.float32),
            pltpu.VMEM((_ROWS_PER_W, DIM), jnp.float32),
        ] + [pltpu.SemaphoreType.DMA] * _NBUF,
        compiler_params=pltpu.CompilerParams(use_tc_tiling_on_sc=False),
    )
    return run(x_tiles, table)
